# Initial kernel scaffold; baseline (speedup 1.0000x reference)
import jax
import jax.numpy as jnp
from jax.experimental import pallas as pl


def kernel(length_q, length_k, embeddings_table):
    raise NotImplementedError("write your pallas kernel here")


# speedup vs baseline: 5.2991x; 5.2991x over previous
import functools

import jax
import jax.numpy as jnp
from jax import lax
from jax.experimental import pallas as pl
from jax.experimental.pallas import tpu as pltpu
from jax.experimental.pallas import tpu_sc as plsc

_MAXP = 128
_D = 64
_LQ = 2048
_LK = 2048

_NC = 2
_NS = 16
_RB = _LQ // (_NC * _NS // 2)
_CB = _LK // 2
_SPAN = _RB + _CB
_IDX_CHUNK = 128
_NCHUNK = _SPAN // _IDX_CHUNK
_LANE_CHUNKS = _SPAN // 16
_GRP = 16


def _rp_body(table_hbm, delta_hbm, out_hbm, idx_v, span_v, delta_v, sem):
    wid = lax.axis_index("s") * _NC + lax.axis_index("c")
    i0 = (wid // 2) * _RB
    j0 = (wid % 2) * _CB

    pltpu.sync_copy(delta_hbm, delta_v)
    delta = delta_v[...]

    base = j0 - i0 - (_RB - 1)

    def fill_idx(u, carry):
        v = lax.iota(jnp.int32, 16) + (u * 16 + base) + delta
        v = jnp.minimum(jnp.maximum(v, -_MAXP), _MAXP) + _MAXP
        idx_v[pl.ds(u * 16, 16)] = v
        return carry

    lax.fori_loop(0, _LANE_CHUNKS, fill_idx, 0)

    gathers = [
        pltpu.async_copy(
            table_hbm.at[idx_v.at[pl.ds(ci * _IDX_CHUNK, _IDX_CHUNK)]],
            span_v.at[pl.ds(ci * _IDX_CHUNK, _IDX_CHUNK)],
            sem,
        )
        for ci in range(_NCHUNK)
    ]
    for g in gathers:
        g.wait()

    def write_group(g, carry):
        r0 = g * _GRP
        copies = [
            pltpu.async_copy(
                span_v.at[pl.ds((_RB - 1) - (r0 + k), _CB)],
                out_hbm.at[i0 + r0 + k, pl.ds(j0, _CB)],
                sem,
            )
            for k in range(_GRP)
        ]
        for c in copies:
            c.wait()
        return carry

    lax.fori_loop(0, _RB // _GRP, write_group, 0)


_rp_call = functools.partial(
    pl.kernel,
    mesh=plsc.VectorSubcoreMesh(core_axis_name="c", subcore_axis_name="s"),
    out_type=jax.ShapeDtypeStruct((_LQ, _LK, _D), jnp.float32),
    scratch_types=[
        pltpu.VMEM((_SPAN,), jnp.int32),
        pltpu.VMEM((_SPAN, _D), jnp.float32),
        pltpu.VMEM((16,), jnp.int32),
        pltpu.SemaphoreType.DMA,
    ],
    compiler_params=pltpu.CompilerParams(use_tc_tiling_on_sc=False),
)(_rp_body)


def kernel(length_q, length_k, embeddings_table):
    delta = jnp.full((16,), 0, jnp.int32) + (
        jnp.asarray(length_k, jnp.int32) - jnp.asarray(length_q, jnp.int32))
    return _rp_call(embeddings_table, delta)


# speedup vs baseline: 5.3052x; 1.0011x over previous
import functools

import jax
import jax.numpy as jnp
from jax import lax
from jax.experimental import pallas as pl
from jax.experimental.pallas import tpu as pltpu
from jax.experimental.pallas import tpu_sc as plsc

_MAXP = 128
_D = 64
_LQ = 2048
_LK = 2048

_NC = 2
_NS = 16
_RB = _LQ // (_NC * _NS // 2)
_CB = _LK // 2
_SPAN = _RB + _CB
_IDX_CHUNK = 128
_NCHUNK = _SPAN // _IDX_CHUNK
_LANE_CHUNKS = _SPAN // 16
_GRP = 16


def _rp_body(table_hbm, delta_hbm, out_hbm, idx_v, span_v, delta_v, sem):
    wid = lax.axis_index("s") * _NC + lax.axis_index("c")
    i0 = (wid // 2) * _RB
    j0 = (wid % 2) * _CB

    pltpu.sync_copy(delta_hbm, delta_v)
    delta = delta_v[...]

    base = j0 - i0 - (_RB - 1)

    def fill_idx(u, carry):
        v = lax.iota(jnp.int32, 16) + (u * 16 + base) + delta
        v = jnp.minimum(jnp.maximum(v, -_MAXP), _MAXP) + _MAXP
        idx_v[pl.ds(u * 16, 16)] = v
        return carry

    lax.fori_loop(0, _LANE_CHUNKS, fill_idx, 0)

    gathers = [
        pltpu.async_copy(
            table_hbm.at[idx_v.at[pl.ds(ci * _IDX_CHUNK, _IDX_CHUNK)]],
            span_v.at[pl.ds(ci * _IDX_CHUNK, _IDX_CHUNK)],
            sem,
        )
        for ci in range(_NCHUNK)
    ]
    for g in gathers:
        g.wait()

    copies = [
        pltpu.async_copy(
            span_v.at[pl.ds((_RB - 1) - r, _CB)],
            out_hbm.at[i0 + r, pl.ds(j0, _CB)],
            sem,
        )
        for r in range(_RB)
    ]
    for c in copies:
        c.wait()


_rp_call = functools.partial(
    pl.kernel,
    mesh=plsc.VectorSubcoreMesh(core_axis_name="c", subcore_axis_name="s"),
    out_type=jax.ShapeDtypeStruct((_LQ, _LK, _D), jnp.float32),
    scratch_types=[
        pltpu.VMEM((_SPAN,), jnp.int32),
        pltpu.VMEM((_SPAN, _D), jnp.float32),
        pltpu.VMEM((16,), jnp.int32),
        pltpu.SemaphoreType.DMA,
    ],
    compiler_params=pltpu.CompilerParams(use_tc_tiling_on_sc=False),
)(_rp_body)


def kernel(length_q, length_k, embeddings_table):
    delta = jnp.full((16,), 0, jnp.int32) + (
        jnp.asarray(length_k, jnp.int32) - jnp.asarray(length_q, jnp.int32))
    return _rp_call(embeddings_table, delta)
